# baseline (speedup 1.0000x reference)
import jax
import jax.numpy as jnp
from jax.experimental import pallas as pl

PROB_DO = 0.5
BLOCK_ROWS = 1024


def _body(c_ref, x_ref, o_ref):
    x = x_ref[...]
    o_ref[...] = jnp.maximum(x, x * c_ref[...])


def kernel(x):
    batch, width = x.shape
    mask = jax.random.uniform(jax.random.key(1), (batch,)) < PROB_DO
    c = (1.0 - mask.astype(x.dtype))[:, None]

    grid = (batch // BLOCK_ROWS,)
    return pl.pallas_call(
        _body,
        grid=grid,
        in_specs=[
            pl.BlockSpec((BLOCK_ROWS, 1), lambda i: (i, 0)),
            pl.BlockSpec((BLOCK_ROWS, width), lambda i: (i, 0)),
        ],
        out_specs=pl.BlockSpec((BLOCK_ROWS, width), lambda i: (i, 0)),
        out_shape=jax.ShapeDtypeStruct((batch, width), x.dtype),
    )(c, x)


# speedup vs baseline: 1.7241x; 1.7241x over previous
import jax
import jax.numpy as jnp
from jax.experimental import pallas as pl

PROB_DO = 0.5
BLOCK_ROWS = 1024


def _body(c_ref, x_ref, o_ref):
    x = x_ref[...]
    o_ref[...] = jnp.maximum(x, x * c_ref[...])


def kernel(x):
    batch, width = x.shape
    with jax.ensure_compile_time_eval():
        mask = jax.random.uniform(jax.random.key(1), (batch,)) < PROB_DO
        c = (1.0 - mask.astype(x.dtype))[:, None]
    grid = (batch // BLOCK_ROWS,)
    return pl.pallas_call(
        _body,
        grid=grid,
        in_specs=[
            pl.BlockSpec((BLOCK_ROWS, 1), lambda i: (i, 0)),
            pl.BlockSpec((BLOCK_ROWS, width), lambda i: (i, 0)),
        ],
        out_specs=pl.BlockSpec((BLOCK_ROWS, width), lambda i: (i, 0)),
        out_shape=jax.ShapeDtypeStruct((batch, width), x.dtype),
    )(c, x)
